# baseline (speedup 1.0000x reference)
import functools

import jax
import jax.numpy as jnp
from jax import lax
from jax.experimental import pallas as pl
from jax.experimental.pallas import tpu as pltpu
from jax.experimental.pallas import tpu_sc as plsc

_NA = 3
_NO = 85
_NCORES = 2
_NSUB = 16
_NW = _NCORES * _NSUB
_SB = 512


def _make_sc_transpose(n_slab, s):
    n_chunks = n_slab * (s // _SB)
    per_w = n_chunks // _NW
    chunk_words = _SB * _NO
    mesh = plsc.VectorSubcoreMesh(core_axis_name="c", subcore_axis_name="s")

    @functools.partial(
        pl.kernel,
        out_type=jax.ShapeDtypeStruct((n_slab * s * _NO,), jnp.float32),
        mesh=mesh,
        scratch_types=[
            pltpu.VMEM((_NO, _SB), jnp.float32),
            pltpu.VMEM((chunk_words,), jnp.float32),
        ],
        compiler_params=pltpu.CompilerParams(needs_layout_passes=False),
    )
    def sc_kernel(x_hbm, out_hbm, in_v, out_v):
        wid = lax.axis_index("s") * _NCORES + lax.axis_index("c")
        scaled_iota = lax.iota(jnp.int32, 16) * _NO

        @pl.loop(0, per_w)
        def _task(t):
            chunk = wid * per_w + t
            n = chunk // (s // _SB)
            j = chunk % (s // _SB)
            pltpu.sync_copy(x_hbm.at[n, :, pl.ds(j * _SB, _SB)], in_v)

            @pl.loop(0, _NO)
            def _row(cc):
                @pl.loop(0, _SB // 16, unroll=8)
                def _blk(ub):
                    u = ub * 16
                    v = in_v[cc, pl.ds(u, 16)]
                    idx = scaled_iota + (u * _NO + cc)
                    plsc.store_scatter(out_v, [idx], v)

            pltpu.sync_copy(out_v, out_hbm.at[pl.ds(chunk * chunk_words, chunk_words)])

    return sc_kernel


def kernel(p):
    bs, c, ny, nx = p.shape
    s = ny * nx
    n_slab = bs * _NA
    x = p.reshape(n_slab, _NO, s)
    out = _make_sc_transpose(n_slab, s)(x)
    return out.reshape(bs, _NA, ny, nx, _NO)


# speedup vs baseline: 1.1342x; 1.1342x over previous
import functools

import jax
import jax.numpy as jnp
from jax import lax
from jax.experimental import pallas as pl
from jax.experimental.pallas import tpu as pltpu
from jax.experimental.pallas import tpu_sc as plsc

_NA = 3
_NO = 85
_NCORES = 2
_NSUB = 16
_NW = _NCORES * _NSUB
_SB = 256


def _make_sc_transpose(n_slab, s):
    cps = s // _SB
    n_chunks = n_slab * cps
    per_w = n_chunks // _NW
    chunk_words = _SB * _NO
    mesh = plsc.VectorSubcoreMesh(core_axis_name="c", subcore_axis_name="s")

    @functools.partial(
        pl.kernel,
        out_type=jax.ShapeDtypeStruct((n_slab * s * _NO,), jnp.float32),
        mesh=mesh,
        scratch_types=[
            pltpu.VMEM((_NO, _SB), jnp.float32),
            pltpu.VMEM((_NO, _SB), jnp.float32),
            pltpu.VMEM((chunk_words,), jnp.float32),
            pltpu.VMEM((chunk_words,), jnp.float32),
            pltpu.SemaphoreType.DMA,
            pltpu.SemaphoreType.DMA,
            pltpu.SemaphoreType.DMA,
            pltpu.SemaphoreType.DMA,
        ],
        compiler_params=pltpu.CompilerParams(needs_layout_passes=False),
    )
    def sc_kernel(x_hbm, out_hbm, in_v0, in_v1, out_v0, out_v1,
                  si0, si1, so0, so1):
        wid = lax.axis_index("s") * _NCORES + lax.axis_index("c")
        scaled_iota = lax.iota(jnp.int32, 16) * _NO
        in_bufs = (in_v0, in_v1)
        out_bufs = (out_v0, out_v1)
        in_sems = (si0, si1)
        out_sems = (so0, so1)

        def in_src(t):
            chunk = wid * per_w + t
            n = chunk // cps
            j = chunk % cps
            return x_hbm.at[n, :, pl.ds(j * _SB, _SB)]

        def out_dst(t):
            chunk = wid * per_w + t
            return out_hbm.at[pl.ds(chunk * chunk_words, chunk_words)]

        pltpu.async_copy(in_src(0), in_bufs[0], in_sems[0])

        @pl.loop(0, per_w // 2)
        def _pair(tp):
            for b in range(2):
                t = tp * 2 + b
                iv, ov = in_bufs[b], out_bufs[b]

                @pl.when(t + 1 < per_w)
                def _prefetch():
                    pltpu.async_copy(in_src(t + 1), in_bufs[1 - b],
                                     in_sems[1 - b])

                pltpu.make_async_copy(in_src(t), iv, in_sems[b]).wait()

                @pl.when(t >= 2)
                def _drain():
                    pltpu.make_async_copy(ov, out_dst(t - 2),
                                          out_sems[b]).wait()

                @pl.loop(0, _NO)
                def _row(cc):
                    @pl.loop(0, _SB // 16, unroll=16)
                    def _blk(ub):
                        u = ub * 16
                        v = iv[cc, pl.ds(u, 16)]
                        idx = scaled_iota + (u * _NO + cc)
                        plsc.store_scatter(ov, [idx], v)

                pltpu.async_copy(ov, out_dst(t), out_sems[b])

        pltpu.make_async_copy(out_bufs[0], out_dst(per_w - 2),
                              out_sems[0]).wait()
        pltpu.make_async_copy(out_bufs[1], out_dst(per_w - 1),
                              out_sems[1]).wait()

    return sc_kernel


def kernel(p):
    bs, c, ny, nx = p.shape
    s = ny * nx
    n_slab = bs * _NA
    x = p.reshape(n_slab, _NO, s)
    out = _make_sc_transpose(n_slab, s)(x)
    return out.reshape(bs, _NA, ny, nx, _NO)


# speedup vs baseline: 1.3361x; 1.1780x over previous
import functools

import jax
import jax.numpy as jnp
from jax import lax
from jax.experimental import pallas as pl
from jax.experimental.pallas import tpu as pltpu
from jax.experimental.pallas import tpu_sc as plsc

_NA = 3
_NO = 85
_NCORES = 2
_NSUB = 16
_NW = _NCORES * _NSUB
_SB = 256


def _make_sc_transpose(n_slab, s):
    cps = s // _SB
    n_chunks = n_slab * cps
    per_w = n_chunks // _NW
    chunk_words = _SB * _NO
    mesh = plsc.VectorSubcoreMesh(core_axis_name="c", subcore_axis_name="s")

    @functools.partial(
        pl.kernel,
        out_type=jax.ShapeDtypeStruct((n_slab * s * _NO,), jnp.float32),
        mesh=mesh,
        scratch_types=[
            pltpu.VMEM((_NO, _SB), jnp.float32),
            pltpu.VMEM((_NO, _SB), jnp.float32),
            pltpu.VMEM((chunk_words,), jnp.float32),
            pltpu.VMEM((chunk_words,), jnp.float32),
            pltpu.SemaphoreType.DMA,
            pltpu.SemaphoreType.DMA,
            pltpu.SemaphoreType.DMA,
            pltpu.SemaphoreType.DMA,
        ],
        compiler_params=pltpu.CompilerParams(needs_layout_passes=False),
    )
    def sc_kernel(x_hbm, out_hbm, in_v0, in_v1, out_v0, out_v1,
                  si0, si1, so0, so1):
        wid = lax.axis_index("s") * _NCORES + lax.axis_index("c")
        scaled_iota = lax.iota(jnp.int32, 16) * _NO
        in_bufs = (in_v0, in_v1)
        out_bufs = (out_v0, out_v1)
        in_sems = (si0, si1)
        out_sems = (so0, so1)

        def in_src(t):
            chunk = wid * per_w + t
            n = chunk // cps
            j = chunk % cps
            return x_hbm.at[n, :, pl.ds(j * _SB, _SB)]

        def out_dst(t):
            chunk = wid * per_w + t
            return out_hbm.at[pl.ds(chunk * chunk_words, chunk_words)]

        pltpu.async_copy(in_src(0), in_bufs[0], in_sems[0])

        @pl.loop(0, per_w // 2)
        def _pair(tp):
            for b in range(2):
                t = tp * 2 + b
                iv, ov = in_bufs[b], out_bufs[b]

                @pl.when(t + 1 < per_w)
                def _prefetch():
                    pltpu.async_copy(in_src(t + 1), in_bufs[1 - b],
                                     in_sems[1 - b])

                pltpu.make_async_copy(in_src(t), iv, in_sems[b]).wait()

                @pl.when(t >= 2)
                def _drain():
                    pltpu.make_async_copy(ov, out_dst(t - 2),
                                          out_sems[b]).wait()

                @pl.loop(0, _NO)
                def _row(cc):
                    vs = [iv[cc, pl.ds(ub * 16, 16)]
                          for ub in range(_SB // 16)]
                    for ub in range(_SB // 16):
                        idx = scaled_iota + (ub * 16 * _NO + cc)
                        plsc.store_scatter(ov, [idx], vs[ub])

                pltpu.async_copy(ov, out_dst(t), out_sems[b])

        pltpu.make_async_copy(out_bufs[0], out_dst(per_w - 2),
                              out_sems[0]).wait()
        pltpu.make_async_copy(out_bufs[1], out_dst(per_w - 1),
                              out_sems[1]).wait()

    return sc_kernel


def kernel(p):
    bs, c, ny, nx = p.shape
    s = ny * nx
    n_slab = bs * _NA
    x = p.reshape(n_slab, _NO, s)
    out = _make_sc_transpose(n_slab, s)(x)
    return out.reshape(bs, _NA, ny, nx, _NO)


# speedup vs baseline: 1.3409x; 1.0036x over previous
import functools

import jax
import jax.numpy as jnp
from jax import lax
from jax.experimental import pallas as pl
from jax.experimental.pallas import tpu as pltpu
from jax.experimental.pallas import tpu_sc as plsc

_NA = 3
_NO = 85
_NCORES = 2
_NSUB = 16
_NW = _NCORES * _NSUB
_SB = 512
_HB = _SB // 4


def _make_sc_transpose(n_slab, s):
    cps = s // _SB
    n_chunks = n_slab * cps
    per_w = n_chunks // _NW
    chunk_words = _SB * _NO
    q_words = _HB * _NO
    qpw = _SB // _HB
    mesh = plsc.VectorSubcoreMesh(core_axis_name="c", subcore_axis_name="s")

    @functools.partial(
        pl.kernel,
        out_type=jax.ShapeDtypeStruct((n_slab * s * _NO,), jnp.float32),
        mesh=mesh,
        scratch_types=[
            pltpu.VMEM((_NO, _SB), jnp.float32),
            pltpu.VMEM((_NO, _SB), jnp.float32),
            pltpu.VMEM((q_words,), jnp.float32),
            pltpu.VMEM((q_words,), jnp.float32),
            pltpu.SemaphoreType.DMA,
            pltpu.SemaphoreType.DMA,
            pltpu.SemaphoreType.DMA,
            pltpu.SemaphoreType.DMA,
        ],
        compiler_params=pltpu.CompilerParams(needs_layout_passes=False),
    )
    def sc_kernel(x_hbm, out_hbm, in_v0, in_v1, out_v0, out_v1,
                  si0, si1, so0, so1):
        wid = lax.axis_index("s") * _NCORES + lax.axis_index("c")
        scaled_iota = lax.iota(jnp.int32, 16) * _NO
        in_bufs = (in_v0, in_v1)
        out_bufs = (out_v0, out_v1)
        in_sems = (si0, si1)
        out_sems = (so0, so1)

        def in_src(t):
            chunk = wid * per_w + t
            n = chunk // cps
            j = chunk % cps
            return x_hbm.at[n, :, pl.ds(j * _SB, _SB)]

        def out_dst(k):
            return out_hbm.at[pl.ds(wid * per_w * chunk_words + k * q_words,
                                    q_words)]

        pltpu.async_copy(in_src(0), in_bufs[0], in_sems[0])

        @pl.loop(0, per_w // 2)
        def _pair(tp):
            for b in range(2):
                t = tp * 2 + b
                iv = in_bufs[b]

                @pl.when(t + 1 < per_w)
                def _prefetch():
                    pltpu.async_copy(in_src(t + 1), in_bufs[1 - b],
                                     in_sems[1 - b])

                pltpu.make_async_copy(in_src(t), iv, in_sems[b]).wait()

                for h in range(qpw):
                    ov = out_bufs[h % 2]
                    kq = t * qpw + h

                    @pl.when(kq >= 2)
                    def _drain():
                        pltpu.make_async_copy(ov, out_dst(kq - 2),
                                              out_sems[h % 2]).wait()

                    @pl.loop(0, _NO)
                    def _row(cc):
                        vs = [iv[cc, pl.ds(h * _HB + ub * 16, 16)]
                              for ub in range(_HB // 16)]
                        for ub in range(_HB // 16):
                            idx = scaled_iota + (ub * 16 * _NO + cc)
                            plsc.store_scatter(ov, [idx], vs[ub])

                    pltpu.async_copy(ov, out_dst(kq), out_sems[h % 2])

        n_q = per_w * qpw
        pltpu.make_async_copy(out_bufs[0], out_dst(n_q - 2),
                              out_sems[0]).wait()
        pltpu.make_async_copy(out_bufs[1], out_dst(n_q - 1),
                              out_sems[1]).wait()

    return sc_kernel


def kernel(p):
    bs, c, ny, nx = p.shape
    s = ny * nx
    n_slab = bs * _NA
    x = p.reshape(n_slab, _NO, s)
    out = _make_sc_transpose(n_slab, s)(x)
    return out.reshape(bs, _NA, ny, nx, _NO)


# speedup vs baseline: 1.5365x; 1.1459x over previous
import functools

import jax
import jax.numpy as jnp
from jax import lax
from jax.experimental import pallas as pl
from jax.experimental.pallas import tpu as pltpu
from jax.experimental.pallas import tpu_sc as plsc

_NA = 3
_NO = 85
_NCORES = 2
_NSUB = 16
_NW = _NCORES * _NSUB
_SB = 512
_HB = _SB // 4


def _make_sc_transpose(n_slab, s):
    cps = s // _SB
    n_chunks = n_slab * cps
    per_w = n_chunks // _NW
    chunk_words = _SB * _NO
    q_words = _HB * _NO
    qpw = _SB // _HB
    mesh = plsc.VectorSubcoreMesh(core_axis_name="c", subcore_axis_name="s")

    @functools.partial(
        pl.kernel,
        out_type=jax.ShapeDtypeStruct((n_slab * s * _NO,), jnp.float32),
        mesh=mesh,
        scratch_types=[
            pltpu.VMEM((_NO, _SB), jnp.float32),
            pltpu.VMEM((_NO, _SB), jnp.float32),
            pltpu.VMEM((q_words,), jnp.float32),
            pltpu.VMEM((q_words,), jnp.float32),
            pltpu.SemaphoreType.DMA,
            pltpu.SemaphoreType.DMA,
            pltpu.SemaphoreType.DMA,
            pltpu.SemaphoreType.DMA,
        ],
        compiler_params=pltpu.CompilerParams(needs_layout_passes=False),
    )
    def sc_kernel(x_hbm, out_hbm, in_v0, in_v1, out_v0, out_v1,
                  si0, si1, so0, so1):
        wid = lax.axis_index("s") * _NCORES + lax.axis_index("c")
        scaled_iota = lax.iota(jnp.int32, 16) * _NO
        in_bufs = (in_v0, in_v1)
        out_bufs = (out_v0, out_v1)
        in_sems = (si0, si1)
        out_sems = (so0, so1)

        def in_src(t):
            chunk = wid * per_w + t
            n = chunk // cps
            j = chunk % cps
            return x_hbm.at[n, :, pl.ds(j * _SB, _SB)]

        def out_dst(k):
            return out_hbm.at[pl.ds(wid * per_w * chunk_words + k * q_words,
                                    q_words)]

        pltpu.async_copy(in_src(0), in_bufs[0], in_sems[0])

        @pl.loop(0, per_w // 2)
        def _pair(tp):
            for b in range(2):
                t = tp * 2 + b
                iv = in_bufs[b]

                @pl.when(t + 1 < per_w)
                def _prefetch():
                    pltpu.async_copy(in_src(t + 1), in_bufs[1 - b],
                                     in_sems[1 - b])

                pltpu.make_async_copy(in_src(t), iv, in_sems[b]).wait()

                for h in range(qpw):
                    ov = out_bufs[h % 2]
                    kq = t * qpw + h

                    @pl.when(kq >= 2)
                    def _drain():
                        pltpu.make_async_copy(ov, out_dst(kq - 2),
                                              out_sems[h % 2]).wait()

                    @pl.loop(0, _NO)
                    def _row(cc):
                        vs = [iv[cc, pl.ds(h * _HB + ub * 16, 16)]
                              for ub in range(_HB // 16)]
                        for ub in range(_HB // 16):
                            idx = scaled_iota + (ub * 16 * _NO + cc)
                            plsc.store_scatter(ov, [idx], vs[ub])

                    pltpu.async_copy(ov, out_dst(kq), out_sems[h % 2])

        n_q = per_w * qpw
        pltpu.make_async_copy(out_bufs[0], out_dst(n_q - 2),
                              out_sems[0]).wait()
        pltpu.make_async_copy(out_bufs[1], out_dst(n_q - 1),
                              out_sems[1]).wait()

    return sc_kernel


_SC_SLABS = 16
_NB = 8


def _transpose_body(in_ref, out_ref):
    out_ref[...] = jnp.transpose(in_ref[...], (0, 2, 1))


def kernel(p):
    bs, c, ny, nx = p.shape
    s = ny * nx
    n_slab = bs * _NA
    x = p.reshape(n_slab, _NO, s)
    sc_flat = _make_sc_transpose(_SC_SLABS, s)(x)
    n_tc = n_slab - _SC_SLABS
    tc = pl.pallas_call(
        _transpose_body,
        grid=(n_tc // _NB,),
        in_specs=[pl.BlockSpec((_NB, _NO, s),
                               lambda i: (i + _SC_SLABS // _NB, 0, 0))],
        out_specs=pl.BlockSpec((_NB, s, _NO), lambda i: (i, 0, 0)),
        out_shape=jax.ShapeDtypeStruct((n_tc, s, _NO), jnp.float32),
    )(x)
    out = jnp.concatenate([sc_flat.reshape(_SC_SLABS, s, _NO), tc], axis=0)
    return out.reshape(bs, _NA, ny, nx, _NO)
